# baseline (speedup 1.0000x reference)
import jax
import jax.numpy as jnp
from jax import lax
from jax.experimental import pallas as pl
from jax.experimental.pallas import tpu as pltpu
from jax.experimental.pallas import tpu_sc as plsc

VOCAB = 100000
EMBED = 16
SEQ = 2048
LANES = 128

_NUM_CORES = 2
_NUM_SUBCORES = 16
_NW = _NUM_CORES * _NUM_SUBCORES
_B_PER_W = SEQ // _NW
_NB = 8

V_TILE = 1024


def _sc_gather_body(tokens_hbm, wt_hbm, out_hbm,
                    tok_s, bufs, out_v, sems):
    wid = lax.axis_index("s") * _NUM_CORES + lax.axis_index("c")
    base = wid * _B_PER_W
    pltpu.sync_copy(tokens_hbm.at[pl.ds(base, _B_PER_W)], tok_s)
    rows = lax.iota(jnp.int32, 16)
    for g in range(_B_PER_W + _NB):
        if g >= _NB:
            gp = g - _NB
            pltpu.make_async_copy(
                wt_hbm.at[:, pl.ds(0, LANES)],
                bufs.at[gp % _NB],
                sems.at[gp % _NB],
            ).wait()
            lane = tok_s[pl.ds((gp // 16) * 16, 16)][gp % 16] & (LANES - 1)
            vals = plsc.load_gather(
                bufs,
                [jnp.full((16,), gp % _NB, jnp.int32),
                 rows,
                 jnp.full((16,), lane, jnp.int32)],
            )
            out_v[gp, :EMBED] = vals
        if g < _B_PER_W:
            t = tok_s[pl.ds((g // 16) * 16, 16)][g % 16]
            col = pl.multiple_of((t >> 7) * LANES, LANES)
            pltpu.make_async_copy(
                wt_hbm.at[:, pl.ds(col, LANES)],
                bufs.at[g % _NB],
                sems.at[g % _NB],
            ).start()
    pltpu.sync_copy(out_v, out_hbm.at[pl.ds(base, _B_PER_W)])


def _sc_gather(tokens, wt):
    k = pl.kernel(
        _sc_gather_body,
        mesh=plsc.VectorSubcoreMesh(core_axis_name="c", subcore_axis_name="s"),
        out_type=jax.ShapeDtypeStruct((SEQ, LANES), jnp.float32),
        scratch_types=[
            pltpu.VMEM((_B_PER_W,), jnp.int32),
            pltpu.VMEM((_NB, EMBED, LANES), jnp.float32),
            pltpu.VMEM((_B_PER_W, LANES), jnp.float32),
            pltpu.SemaphoreType.DMA((_NB,)),
        ],
        compiler_params=pltpu.CompilerParams(
            use_tc_tiling_on_sc=True, needs_layout_passes=False),
    )
    return k(tokens, wt)


def _mm_body(wt_ref, values_ref, b_ref, out_ref):
    out_ref[...] = lax.dot_general(
        wt_ref[...], values_ref[:, :EMBED],
        dimension_numbers=(((0,), (1,)), ((), ())),
        preferred_element_type=jnp.float32,
    ) + b_ref[...][:, None]


def _project(wt, values, bias):
    return pl.pallas_call(
        _mm_body,
        grid=(pl.cdiv(VOCAB, V_TILE),),
        in_specs=[
            pl.BlockSpec((EMBED, V_TILE), lambda i: (0, i)),
            pl.BlockSpec((SEQ, LANES), lambda i: (0, 0)),
            pl.BlockSpec((V_TILE,), lambda i: (i,)),
        ],
        out_specs=pl.BlockSpec((V_TILE, SEQ), lambda i: (i, 0)),
        out_shape=jax.ShapeDtypeStruct((VOCAB, SEQ), jnp.float32),
    )(wt, values, bias)


def kernel(tokens, weight, bias):
    wt = weight.T
    values = _sc_gather(tokens.astype(jnp.int32), wt)
    return _project(wt, values, bias).T


# speedup vs baseline: 1.0041x; 1.0041x over previous
import jax
import jax.numpy as jnp
from jax import lax
from jax.experimental import pallas as pl
from jax.experimental.pallas import tpu as pltpu
from jax.experimental.pallas import tpu_sc as plsc

VOCAB = 100000
EMBED = 16
SEQ = 2048
LANES = 128

_NUM_CORES = 2
_NUM_SUBCORES = 16
_NW = _NUM_CORES * _NUM_SUBCORES
_B_PER_W = SEQ // _NW
_NB = 8

V_TILE = 3072


def _sc_gather_body(tokens_hbm, wt_hbm, out_hbm,
                    tok_s, bufs, out_v, sems):
    wid = lax.axis_index("s") * _NUM_CORES + lax.axis_index("c")
    base = wid * _B_PER_W
    pltpu.sync_copy(tokens_hbm.at[pl.ds(base, _B_PER_W)], tok_s)
    rows = lax.iota(jnp.int32, 16)
    for g in range(_B_PER_W + _NB):
        if g >= _NB:
            gp = g - _NB
            pltpu.make_async_copy(
                wt_hbm.at[:, pl.ds(0, LANES)],
                bufs.at[gp % _NB],
                sems.at[gp % _NB],
            ).wait()
            lane = tok_s[pl.ds((gp // 16) * 16, 16)][gp % 16] & (LANES - 1)
            vals = plsc.load_gather(
                bufs,
                [jnp.full((16,), gp % _NB, jnp.int32),
                 rows,
                 jnp.full((16,), lane, jnp.int32)],
            )
            out_v[gp, :EMBED] = vals
        if g < _B_PER_W:
            t = tok_s[pl.ds((g // 16) * 16, 16)][g % 16]
            col = pl.multiple_of((t >> 7) * LANES, LANES)
            pltpu.make_async_copy(
                wt_hbm.at[:, pl.ds(col, LANES)],
                bufs.at[g % _NB],
                sems.at[g % _NB],
            ).start()
    pltpu.sync_copy(out_v, out_hbm.at[pl.ds(base, _B_PER_W)])


def _sc_gather(tokens, wt):
    k = pl.kernel(
        _sc_gather_body,
        mesh=plsc.VectorSubcoreMesh(core_axis_name="c", subcore_axis_name="s"),
        out_type=jax.ShapeDtypeStruct((SEQ, LANES), jnp.float32),
        scratch_types=[
            pltpu.VMEM((_B_PER_W,), jnp.int32),
            pltpu.VMEM((_NB, EMBED, LANES), jnp.float32),
            pltpu.VMEM((_B_PER_W, LANES), jnp.float32),
            pltpu.SemaphoreType.DMA((_NB,)),
        ],
        compiler_params=pltpu.CompilerParams(
            use_tc_tiling_on_sc=True, needs_layout_passes=False),
    )
    return k(tokens, wt)


def _mm_body(wt_ref, values_ref, b_ref, out_ref):
    out_ref[...] = lax.dot_general(
        wt_ref[...], values_ref[:, :EMBED],
        dimension_numbers=(((0,), (1,)), ((), ())),
        preferred_element_type=jnp.float32,
    ) + b_ref[...][:, None]


def _project(wt, values, bias):
    return pl.pallas_call(
        _mm_body,
        grid=(pl.cdiv(VOCAB, V_TILE),),
        in_specs=[
            pl.BlockSpec((EMBED, V_TILE), lambda i: (0, i)),
            pl.BlockSpec((SEQ, LANES), lambda i: (0, 0)),
            pl.BlockSpec((V_TILE,), lambda i: (i,)),
        ],
        out_specs=pl.BlockSpec((V_TILE, SEQ), lambda i: (i, 0)),
        out_shape=jax.ShapeDtypeStruct((VOCAB, SEQ), jnp.float32),
    )(wt, values, bias)


def kernel(tokens, weight, bias):
    wt = weight.T
    values = _sc_gather(tokens.astype(jnp.int32), wt)
    return _project(wt, values, bias).T


# speedup vs baseline: 1.0065x; 1.0023x over previous
import jax
import jax.numpy as jnp
from jax import lax
from jax.experimental import pallas as pl
from jax.experimental.pallas import tpu as pltpu
from jax.experimental.pallas import tpu_sc as plsc

VOCAB = 100000
EMBED = 16
SEQ = 2048
LANES = 128

_NUM_CORES = 2
_NUM_SUBCORES = 16
_NW = _NUM_CORES * _NUM_SUBCORES
_B_PER_W = SEQ // _NW
_NB = 8

V_TILE = 2048


def _sc_gather_body(tokens_hbm, wt_hbm, out_hbm,
                    tok_s, bufs, out_v, sems):
    wid = lax.axis_index("s") * _NUM_CORES + lax.axis_index("c")
    base = wid * _B_PER_W
    pltpu.sync_copy(tokens_hbm.at[pl.ds(base, _B_PER_W)], tok_s)
    rows = lax.iota(jnp.int32, 16)
    for g in range(_B_PER_W + _NB):
        if g >= _NB:
            gp = g - _NB
            pltpu.make_async_copy(
                wt_hbm.at[:, pl.ds(0, LANES)],
                bufs.at[gp % _NB],
                sems.at[gp % _NB],
            ).wait()
            lane = tok_s[pl.ds((gp // 16) * 16, 16)][gp % 16] & (LANES - 1)
            vals = plsc.load_gather(
                bufs,
                [jnp.full((16,), gp % _NB, jnp.int32),
                 rows,
                 jnp.full((16,), lane, jnp.int32)],
            )
            out_v[gp, :EMBED] = vals
        if g < _B_PER_W:
            t = tok_s[pl.ds((g // 16) * 16, 16)][g % 16]
            col = pl.multiple_of((t >> 7) * LANES, LANES)
            pltpu.make_async_copy(
                wt_hbm.at[:, pl.ds(col, LANES)],
                bufs.at[g % _NB],
                sems.at[g % _NB],
            ).start()
    pltpu.sync_copy(out_v, out_hbm.at[pl.ds(base, _B_PER_W)])


def _sc_gather(tokens, wt):
    k = pl.kernel(
        _sc_gather_body,
        mesh=plsc.VectorSubcoreMesh(core_axis_name="c", subcore_axis_name="s"),
        out_type=jax.ShapeDtypeStruct((SEQ, LANES), jnp.float32),
        scratch_types=[
            pltpu.VMEM((_B_PER_W,), jnp.int32),
            pltpu.VMEM((_NB, EMBED, LANES), jnp.float32),
            pltpu.VMEM((_B_PER_W, LANES), jnp.float32),
            pltpu.SemaphoreType.DMA((_NB,)),
        ],
        compiler_params=pltpu.CompilerParams(
            use_tc_tiling_on_sc=True, needs_layout_passes=False),
    )
    return k(tokens, wt)


def _mm_body(wt_ref, values_ref, b_ref, out_ref):
    out_ref[...] = lax.dot_general(
        wt_ref[...], values_ref[:, :EMBED],
        dimension_numbers=(((0,), (1,)), ((), ())),
        preferred_element_type=jnp.float32,
    ) + b_ref[...][:, None]


def _project(wt, values, bias):
    return pl.pallas_call(
        _mm_body,
        grid=(pl.cdiv(VOCAB, V_TILE),),
        in_specs=[
            pl.BlockSpec((EMBED, V_TILE), lambda i: (0, i)),
            pl.BlockSpec((SEQ, LANES), lambda i: (0, 0)),
            pl.BlockSpec((V_TILE,), lambda i: (i,)),
        ],
        out_specs=pl.BlockSpec((V_TILE, SEQ), lambda i: (i, 0)),
        out_shape=jax.ShapeDtypeStruct((VOCAB, SEQ), jnp.float32),
        compiler_params=pltpu.CompilerParams(
            dimension_semantics=("parallel",)),
    )(wt, values, bias)


def kernel(tokens, weight, bias):
    wt = weight.T
    values = _sc_gather(tokens.astype(jnp.int32), wt)
    return _project(wt, values, bias).T


# speedup vs baseline: 1.0138x; 1.0073x over previous
import jax
import jax.numpy as jnp
from jax import lax
from jax.experimental import pallas as pl
from jax.experimental.pallas import tpu as pltpu
from jax.experimental.pallas import tpu_sc as plsc

VOCAB = 100000
EMBED = 16
SEQ = 2048
LANES = 128

_NUM_CORES = 2
_NUM_SUBCORES = 16
_NW = _NUM_CORES * _NUM_SUBCORES
_B_PER_W = SEQ // _NW
_NB = 8

V_TILE = 2048


def _sc_gather_body(tokens_hbm, wt_hbm, out_hbm,
                    tok_s, bufs, out_v, sems):
    wid = lax.axis_index("s") * _NUM_CORES + lax.axis_index("c")
    base = wid * _B_PER_W
    pltpu.sync_copy(tokens_hbm.at[pl.ds(base, _B_PER_W)], tok_s)
    rows = lax.iota(jnp.int32, 16)
    for g in range(_B_PER_W + _NB):
        if g >= _NB:
            gp = g - _NB
            pltpu.make_async_copy(
                wt_hbm.at[:, pl.ds(0, LANES)],
                bufs.at[gp % _NB],
                sems.at[gp % _NB],
            ).wait()
            lane = tok_s[pl.ds((gp // 16) * 16, 16)][gp % 16] & (LANES - 1)
            vals = plsc.load_gather(
                bufs,
                [jnp.full((16,), gp % _NB, jnp.int32),
                 rows,
                 jnp.full((16,), lane, jnp.int32)],
            )
            out_v[gp, :EMBED] = vals
        if g < _B_PER_W:
            t = tok_s[pl.ds((g // 16) * 16, 16)][g % 16]
            col = pl.multiple_of((t >> 7) * LANES, LANES)
            pltpu.make_async_copy(
                wt_hbm.at[:, pl.ds(col, LANES)],
                bufs.at[g % _NB],
                sems.at[g % _NB],
            ).start()
    pltpu.sync_copy(out_v, out_hbm.at[pl.ds(base, _B_PER_W)])


def _sc_gather(tokens, wt):
    k = pl.kernel(
        _sc_gather_body,
        mesh=plsc.VectorSubcoreMesh(core_axis_name="c", subcore_axis_name="s"),
        out_type=jax.ShapeDtypeStruct((SEQ, LANES), jnp.float32),
        scratch_types=[
            pltpu.VMEM((_B_PER_W,), jnp.int32),
            pltpu.VMEM((_NB, EMBED, LANES), jnp.float32),
            pltpu.VMEM((_B_PER_W, LANES), jnp.float32),
            pltpu.SemaphoreType.DMA((_NB,)),
        ],
        compiler_params=pltpu.CompilerParams(
            use_tc_tiling_on_sc=True, needs_layout_passes=False),
    )
    return k(tokens, wt)


def _mm_body(wt_ref, values_ref, b_ref, out_ref):
    out_ref[...] = lax.dot_general(
        wt_ref[...], values_ref[:, :EMBED],
        dimension_numbers=(((0,), (1,)), ((), ())),
        preferred_element_type=jnp.float32,
    ) + b_ref[...][:, None]


def _project(wt, values, bias):
    return pl.pallas_call(
        _mm_body,
        grid=(pl.cdiv(VOCAB, V_TILE),),
        in_specs=[
            pl.BlockSpec((EMBED, V_TILE), lambda i: (0, i)),
            pl.BlockSpec((SEQ, LANES), lambda i: (0, 0)),
            pl.BlockSpec((V_TILE,), lambda i: (i,)),
        ],
        out_specs=pl.BlockSpec((V_TILE, SEQ), lambda i: (i, 0)),
        out_shape=jax.ShapeDtypeStruct((VOCAB, SEQ), jnp.float32),
    )(wt, values, bias)


def kernel(tokens, weight, bias):
    wt = weight.T
    values = _sc_gather(tokens.astype(jnp.int32), wt)
    return _project(wt, values, bias).T
